# baseline (speedup 1.0000x reference)
import functools

import jax
import jax.numpy as jnp
import numpy as np
from jax import lax
from jax.experimental import pallas as pl
from jax.experimental.pallas import tpu as pltpu
from jax.experimental.pallas import tpu_sc as plsc

N_ROWS = 65536
N_COLS = 64
MAGIC = np.float32(12582912.0)

NC = 2
NS = 16
L = 16
NW = NC * NS
ROWS_PER_W = N_ROWS // NW
CH = 256
N_CHUNKS = ROWS_PER_W // CH
GROUPS = CH // L
N_CHAINS = 4

_mesh = plsc.VectorSubcoreMesh(core_axis_name="c", subcore_axis_name="s")


@functools.partial(
    pl.kernel,
    mesh=_mesh,
    out_type=jax.ShapeDtypeStruct((N_COLS, N_ROWS), jnp.float32),
    scratch_types=[
        pltpu.VMEM((N_COLS, CH), jnp.float32),
        pltpu.VMEM((N_COLS, CH), jnp.float32),
        pltpu.VMEM((N_COLS, CH), jnp.float32),
        pltpu.VMEM((N_COLS, CH), jnp.float32),
        pltpu.SemaphoreType.DMA,
        pltpu.SemaphoreType.DMA,
        pltpu.SemaphoreType.DMA,
        pltpu.SemaphoreType.DMA,
    ],
    compiler_params=pltpu.CompilerParams(needs_layout_passes=False),
)
def _dn_quantize(xt_hbm, out_hbm, in0, in1, ou0, ou1, si0, si1, so0, so1):
    wid = lax.axis_index("s") * NC + lax.axis_index("c")
    iota = lax.iota(jnp.int32, L)
    w_col0 = wid * ROWS_PER_W

    def in_slice(t):
        return xt_hbm.at[:, pl.ds(w_col0 + t * CH, CH)]

    def out_slice(t):
        return out_hbm.at[:, pl.ds(w_col0 + t * CH, CH)]

    def compute_chunk(in_buf, out_buf):
        @plsc.parallel_loop(0, GROUPS, unroll=8)
        def _grp(g):
            g0 = g * L
            posv = iota + g0
            m = [jnp.full((L,), -1.0, jnp.float32)] * N_CHAINS
            kb = [jnp.zeros((L,), jnp.int32)] * N_CHAINS
            sm = [jnp.zeros((L,), jnp.float32)] * N_CHAINS
            span = N_COLS // N_CHAINS
            for j in range(N_COLS):
                c = j // span
                v = in_buf[j, pl.ds(g0, L)]
                f = (v + MAGIC) - MAGIC
                out_buf[j, pl.ds(g0, L)] = f
                a = jnp.abs(v - f)
                p = a > m[c]
                m[c] = jnp.where(p, a, m[c])
                kb[c] = jnp.where(p, jnp.full((L,), j, jnp.int32), kb[c])
                sm[c] = sm[c] + f
            mm, kk, ss = m[0], kb[0], sm[0]
            for c in range(1, N_CHAINS):
                p = m[c] > mm
                mm = jnp.where(p, m[c], mm)
                kk = jnp.where(p, kb[c], kk)
                ss = ss + sm[c]
            odd = (ss.astype(jnp.int32) & 1) == 1
            vk = plsc.load_gather(in_buf, [kk, posv])
            fk = (vk + MAGIC) - MAGIC
            fix = jnp.where(vk - fk < 0, jnp.float32(-1.0), jnp.float32(1.0))
            plsc.addupdate_scatter(out_buf, [kk, posv], fix, mask=odd)

    def slot(u, t, in_buf, out_buf, in_sem, out_sem):
        @pl.when(u > 0)
        def _():
            pltpu.make_async_copy(out_buf, out_slice(t - 2), out_sem).wait()

        pltpu.make_async_copy(in_slice(t), in_buf, in_sem).wait()
        compute_chunk(in_buf, out_buf)
        pltpu.async_copy(out_buf, out_slice(t), out_sem)

        @pl.when(u < N_CHUNKS // 2 - 1)
        def _():
            pltpu.async_copy(in_slice(t + 2), in_buf, in_sem)

    pltpu.async_copy(in_slice(0), in0, si0)
    pltpu.async_copy(in_slice(1), in1, si1)

    def pair_body(u, carry):
        slot(u, 2 * u, in0, ou0, si0, so0)
        slot(u, 2 * u + 1, in1, ou1, si1, so1)
        return carry

    lax.fori_loop(0, N_CHUNKS // 2, pair_body, 0)

    last = N_CHUNKS - 2
    pltpu.make_async_copy(ou0, out_slice(last), so0).wait()
    pltpu.make_async_copy(ou1, out_slice(last + 1), so1).wait()


def kernel(x):
    return _dn_quantize(x.T).T


# speedup vs baseline: 1.6412x; 1.6412x over previous
import functools

import jax
import jax.numpy as jnp
import numpy as np
from jax import lax
from jax.experimental import pallas as pl
from jax.experimental.pallas import tpu as pltpu
from jax.experimental.pallas import tpu_sc as plsc

N_ROWS = 65536
N_COLS = 64
MAGIC = np.float32(12582912.0)

NC = 2
NS = 16
L = 16
NW = NC * NS
ROWS_PER_W = N_ROWS // NW
CH = 256
N_CHUNKS = ROWS_PER_W // CH
GROUPS = CH // L
N_CHAINS = 8

_mesh = plsc.VectorSubcoreMesh(core_axis_name="c", subcore_axis_name="s")


@functools.partial(
    pl.kernel,
    mesh=_mesh,
    out_type=jax.ShapeDtypeStruct((N_COLS, N_ROWS), jnp.float32),
    scratch_types=[
        pltpu.VMEM((N_COLS, CH), jnp.float32),
        pltpu.VMEM((N_COLS, CH), jnp.float32),
        pltpu.VMEM((N_COLS, CH), jnp.float32),
        pltpu.VMEM((N_COLS, CH), jnp.float32),
        pltpu.SemaphoreType.DMA,
        pltpu.SemaphoreType.DMA,
        pltpu.SemaphoreType.DMA,
        pltpu.SemaphoreType.DMA,
    ],
    compiler_params=pltpu.CompilerParams(needs_layout_passes=False),
)
def _dn_quantize(xt_hbm, out_hbm, in0, in1, ou0, ou1, si0, si1, so0, so1):
    wid = lax.axis_index("s") * NC + lax.axis_index("c")
    iota = lax.iota(jnp.int32, L)
    w_col0 = wid * ROWS_PER_W

    def in_slice(t):
        return xt_hbm.at[:, pl.ds(w_col0 + t * CH, CH)]

    def out_slice(t):
        return out_hbm.at[:, pl.ds(w_col0 + t * CH, CH)]

    def compute_chunk(in_buf, out_buf):
        @plsc.parallel_loop(0, GROUPS, unroll=4)
        def _grp(g):
            g0 = g * L
            posv = iota + g0
            m = [jnp.full((L,), -1.0, jnp.float32)] * N_CHAINS
            kb = [jnp.zeros((L,), jnp.int32)] * N_CHAINS
            sm = [jnp.zeros((L,), jnp.float32)] * N_CHAINS
            span = N_COLS // N_CHAINS
            for j in range(N_COLS):
                c = j // span
                v = in_buf[j, pl.ds(g0, L)]
                f = (v + MAGIC) - MAGIC
                out_buf[j, pl.ds(g0, L)] = f
                a = jnp.abs(v - f)
                p = a > m[c]
                m[c] = jnp.where(p, a, m[c])
                kb[c] = jnp.where(p, jnp.full((L,), j, jnp.int32), kb[c])
                sm[c] = sm[c] + f
            mm, kk, ss = m[0], kb[0], sm[0]
            for c in range(1, N_CHAINS):
                p = m[c] > mm
                mm = jnp.where(p, m[c], mm)
                kk = jnp.where(p, kb[c], kk)
                ss = ss + sm[c]
            odd = (ss.astype(jnp.int32) & 1) == 1
            vk = plsc.load_gather(in_buf, [kk, posv])
            fk = (vk + MAGIC) - MAGIC
            fix = jnp.where(vk - fk < 0, jnp.float32(-1.0), jnp.float32(1.0))
            plsc.addupdate_scatter(out_buf, [kk, posv], fix, mask=odd)

    def slot(u, t, in_buf, out_buf, in_sem, out_sem):
        @pl.when(u > 0)
        def _():
            pltpu.make_async_copy(out_buf, out_slice(t - 2), out_sem).wait()

        pltpu.make_async_copy(in_slice(t), in_buf, in_sem).wait()
        compute_chunk(in_buf, out_buf)
        pltpu.async_copy(out_buf, out_slice(t), out_sem)

        @pl.when(u < N_CHUNKS // 2 - 1)
        def _():
            pltpu.async_copy(in_slice(t + 2), in_buf, in_sem)

    pltpu.async_copy(in_slice(0), in0, si0)
    pltpu.async_copy(in_slice(1), in1, si1)

    def pair_body(u, carry):
        slot(u, 2 * u, in0, ou0, si0, so0)
        slot(u, 2 * u + 1, in1, ou1, si1, so1)
        return carry

    lax.fori_loop(0, N_CHUNKS // 2, pair_body, 0)

    last = N_CHUNKS - 2
    pltpu.make_async_copy(ou0, out_slice(last), so0).wait()
    pltpu.make_async_copy(ou1, out_slice(last + 1), so1).wait()


def kernel(x):
    return _dn_quantize(x.T).T
